# Initial kernel scaffold; baseline (speedup 1.0000x reference)
import jax
import jax.numpy as jnp
from jax import lax
from jax.experimental import pallas as pl
from jax.experimental.pallas import tpu as pltpu
from jax.experimental.pallas import tpu_sc as plsc

N = 10000
F_IN = 256
HC = 64
NPAD = 10016
STRIPE = NPAD // 16
E = 160000
NC, NS = 2, 16
NW = NC * NS
EPW = 5120
EPAD = EPW * NW
CH = 128
NCHUNK = EPW // CH
BM = 400


def _leaky(x):
    return jnp.where(x >= 0, x, 0.2 * x)


def _dense1_body(x_ref, w_ref, asm_ref, adm_ref, h_ref, as_ref, ad_ref, mx_ref):
    i = pl.program_id(0)
    h = jnp.dot(x_ref[...], w_ref[...], preferred_element_type=jnp.float32)
    a_s = jnp.dot(h, asm_ref[...], preferred_element_type=jnp.float32)
    a_d = jnp.dot(h, adm_ref[...], preferred_element_type=jnp.float32)
    h_ref[...] = h
    as_ref[...] = a_s
    ad_ref[...] = a_d

    @pl.when(i == 0)
    def _():
        mx_ref[...] = jnp.full((2, 16), -3.0e38, jnp.float32)

    upd = jnp.concatenate(
        [jnp.max(a_s, axis=0, keepdims=True), jnp.max(a_d, axis=0, keepdims=True)],
        axis=0,
    )
    mx_ref[...] = jnp.maximum(mx_ref[...], upd)


def _dense1(x, w1, asm, adm):
    return pl.pallas_call(
        _dense1_body,
        grid=(N // BM,),
        in_specs=[
            pl.BlockSpec((BM, F_IN), lambda i: (i, 0)),
            pl.BlockSpec((F_IN, HC), lambda i: (0, 0)),
            pl.BlockSpec((HC, 16), lambda i: (0, 0)),
            pl.BlockSpec((HC, 16), lambda i: (0, 0)),
        ],
        out_specs=[
            pl.BlockSpec((BM, HC), lambda i: (i, 0)),
            pl.BlockSpec((BM, 16), lambda i: (i, 0)),
            pl.BlockSpec((BM, 16), lambda i: (i, 0)),
            pl.BlockSpec((2, 16), lambda i: (0, 0)),
        ],
        out_shape=[
            jax.ShapeDtypeStruct((N, HC), jnp.float32),
            jax.ShapeDtypeStruct((N, 16), jnp.float32),
            jax.ShapeDtypeStruct((N, 16), jnp.float32),
            jax.ShapeDtypeStruct((2, 16), jnp.float32),
        ],
    )(x, w1, asm, adm)


def _make_edge_kernel(nheads):
    mesh = plsc.VectorSubcoreMesh(core_axis_name="c", subcore_axis_name="s")

    def body(h_hbm, as_hbm, ad_hbm, src_hbm, dst_hbm, kv_hbm, z64_hbm, z16_hbm,
             m_out, e_out, sidx, didx, hrows, arows, drows, kv_v, ebuf,
             acc_m, acc_e, sem1, sem2, sem3):
        c = lax.axis_index("c")
        s = lax.axis_index("s")
        wid = s * NC + c
        pltpu.sync_copy(z64_hbm.at[pl.ds(s * STRIPE, STRIPE)],
                        acc_m.at[pl.ds(s * STRIPE, STRIPE)])
        pltpu.sync_copy(z16_hbm.at[pl.ds(s * STRIPE, STRIPE)],
                        acc_e.at[pl.ds(s * STRIPE, STRIPE)])
        pltpu.sync_copy(kv_hbm, kv_v)
        plsc.subcore_barrier()

        kv = kv_v[...]
        iota = lax.iota(jnp.int32, 16)
        half = lax.shift_right_logical(iota, 3)

        def chunk_body(ch, carry):
            base = wid * EPW + ch * CH
            pltpu.sync_copy(src_hbm.at[pl.ds(base, CH)], sidx)
            pltpu.sync_copy(dst_hbm.at[pl.ds(base, CH)], didx)
            cp1 = pltpu.async_copy(h_hbm.at[sidx], hrows, sem1)
            cp2 = pltpu.async_copy(as_hbm.at[sidx], arows, sem2)
            cp3 = pltpu.async_copy(ad_hbm.at[didx], drows, sem3)
            cp1.wait()
            cp2.wait()
            cp3.wait()

            def edge_body(i, carry2):
                a = arows[i, :] + drows[i, :]
                e = jnp.exp(_leaky(a) - kv)
                ebuf[...] = e
                arows[i, :] = e
                for j in range(4):
                    if nheads == 8:
                        patt = half + (2 * j)
                    else:
                        patt = iota * 0
                    eb = plsc.load_gather(ebuf, [patt])
                    hrows[i, pl.ds(16 * j, 16)] = hrows[i, pl.ds(16 * j, 16)] * eb
                return carry2

            lax.fori_loop(0, CH, edge_body, 0)
            pltpu.sync_copy(hrows, acc_m.at[didx], add=True)
            pltpu.sync_copy(arows, acc_e.at[didx], add=True)
            return carry

        lax.fori_loop(0, NCHUNK, chunk_body, 0)
        plsc.subcore_barrier()
        pltpu.sync_copy(acc_m.at[pl.ds(s * STRIPE, STRIPE)],
                        m_out.at[c, pl.ds(s * STRIPE, STRIPE)])
        pltpu.sync_copy(acc_e.at[pl.ds(s * STRIPE, STRIPE)],
                        e_out.at[c, pl.ds(s * STRIPE, STRIPE)])

    return pl.kernel(
        body,
        out_type=[
            jax.ShapeDtypeStruct((NC, NPAD, HC), jnp.float32),
            jax.ShapeDtypeStruct((NC, NPAD, 16), jnp.float32),
        ],
        mesh=mesh,
        scratch_types=[
            pltpu.VMEM((CH,), jnp.int32),
            pltpu.VMEM((CH,), jnp.int32),
            pltpu.VMEM((CH, HC), jnp.float32),
            pltpu.VMEM((CH, 16), jnp.float32),
            pltpu.VMEM((CH, 16), jnp.float32),
            pltpu.VMEM((16,), jnp.float32),
            pltpu.VMEM((16,), jnp.float32),
            pltpu.VMEM_SHARED((NPAD, HC), jnp.float32),
            pltpu.VMEM_SHARED((NPAD, 16), jnp.float32),
            pltpu.SemaphoreType.DMA,
            pltpu.SemaphoreType.DMA,
            pltpu.SemaphoreType.DMA,
        ],
    )


_edge_kernel_h8 = _make_edge_kernel(8)
_edge_kernel_h1 = _make_edge_kernel(1)


def _epi1_body(m_ref, e_ref, h_ref, as_ref, ad_ref, kv_ref, b_ref, r_ref,
               w2_ref, asm_ref, adm_ref, h2_ref, as2_ref, ad2_ref, mx_ref):
    i = pl.program_id(0)
    m = m_ref[...][0] + m_ref[...][1]
    e2 = e_ref[...][0] + e_ref[...][1]
    a = as_ref[...] + ad_ref[...]
    es = jnp.exp(_leaky(a) - kv_ref[...])
    den = jnp.dot(e2 + es, r_ref[...], preferred_element_type=jnp.float32)
    esb = jnp.dot(es, r_ref[...], preferred_element_type=jnp.float32)
    num = m + h_ref[...] * esb
    h1 = num / (den + 1e-16) + b_ref[...]
    h1e = jnp.where(h1 > 0, h1, jnp.exp(h1) - 1.0)
    h2 = jnp.dot(h1e, w2_ref[...], preferred_element_type=jnp.float32)
    a_s2 = jnp.dot(h2, asm_ref[...], preferred_element_type=jnp.float32)
    a_d2 = jnp.dot(h2, adm_ref[...], preferred_element_type=jnp.float32)
    h2_ref[...] = h2
    as2_ref[...] = a_s2
    ad2_ref[...] = a_d2

    @pl.when(i == 0)
    def _():
        mx_ref[...] = jnp.full((2, 16), -3.0e38, jnp.float32)

    upd = jnp.concatenate(
        [jnp.max(a_s2, axis=0, keepdims=True), jnp.max(a_d2, axis=0, keepdims=True)],
        axis=0,
    )
    mx_ref[...] = jnp.maximum(mx_ref[...], upd)


def _epi1(m1, e1, h1, as1, ad1, kv1, b1, r16, w2, asm2, adm2):
    return pl.pallas_call(
        _epi1_body,
        grid=(N // BM,),
        in_specs=[
            pl.BlockSpec((2, BM, HC), lambda i: (0, i, 0)),
            pl.BlockSpec((2, BM, 16), lambda i: (0, i, 0)),
            pl.BlockSpec((BM, HC), lambda i: (i, 0)),
            pl.BlockSpec((BM, 16), lambda i: (i, 0)),
            pl.BlockSpec((BM, 16), lambda i: (i, 0)),
            pl.BlockSpec((1, 16), lambda i: (0, 0)),
            pl.BlockSpec((1, HC), lambda i: (0, 0)),
            pl.BlockSpec((16, HC), lambda i: (0, 0)),
            pl.BlockSpec((HC, HC), lambda i: (0, 0)),
            pl.BlockSpec((HC, 16), lambda i: (0, 0)),
            pl.BlockSpec((HC, 16), lambda i: (0, 0)),
        ],
        out_specs=[
            pl.BlockSpec((BM, HC), lambda i: (i, 0)),
            pl.BlockSpec((BM, 16), lambda i: (i, 0)),
            pl.BlockSpec((BM, 16), lambda i: (i, 0)),
            pl.BlockSpec((2, 16), lambda i: (0, 0)),
        ],
        out_shape=[
            jax.ShapeDtypeStruct((N, HC), jnp.float32),
            jax.ShapeDtypeStruct((N, 16), jnp.float32),
            jax.ShapeDtypeStruct((N, 16), jnp.float32),
            jax.ShapeDtypeStruct((2, 16), jnp.float32),
        ],
    )(m1, e1, h1, as1, ad1, kv1, b1, r16, w2, asm2, adm2)


def _epi2_body(m_ref, e_ref, h_ref, as_ref, ad_ref, kv_ref, b_ref, r_ref,
               out_ref):
    m = m_ref[...][0] + m_ref[...][1]
    e2 = e_ref[...][0] + e_ref[...][1]
    a = as_ref[...] + ad_ref[...]
    es = jnp.exp(_leaky(a) - kv_ref[...])
    den = jnp.dot(e2 + es, r_ref[...], preferred_element_type=jnp.float32)
    esb = jnp.dot(es, r_ref[...], preferred_element_type=jnp.float32)
    num = m + h_ref[...] * esb
    o = num / (den + 1e-16) + b_ref[...]
    mx = jnp.max(o, axis=1, keepdims=True)
    z = o - mx
    lse = jnp.log(jnp.sum(jnp.exp(z), axis=1, keepdims=True))
    out_ref[...] = z - lse


def _epi2(m2, e2, h2, as2, ad2, kv2, b2, r16):
    return pl.pallas_call(
        _epi2_body,
        grid=(N // BM,),
        in_specs=[
            pl.BlockSpec((2, BM, HC), lambda i: (0, i, 0)),
            pl.BlockSpec((2, BM, 16), lambda i: (0, i, 0)),
            pl.BlockSpec((BM, HC), lambda i: (i, 0)),
            pl.BlockSpec((BM, 16), lambda i: (i, 0)),
            pl.BlockSpec((BM, 16), lambda i: (i, 0)),
            pl.BlockSpec((1, 16), lambda i: (0, 0)),
            pl.BlockSpec((1, HC), lambda i: (0, 0)),
            pl.BlockSpec((16, HC), lambda i: (0, 0)),
        ],
        out_specs=pl.BlockSpec((BM, HC), lambda i: (i, 0)),
        out_shape=jax.ShapeDtypeStruct((N, HC), jnp.float32),
    )(m2, e2, h2, as2, ad2, kv2, b2, r16)


def _pack_mats(att_src, att_dst, ch):
    fs = att_src.reshape(HC)
    fd = att_dst.reshape(HC)
    rows = jnp.arange(HC)
    cols = jnp.arange(16)
    sel = (cols[None, :] == (rows[:, None] // ch)).astype(jnp.float32)
    return fs[:, None] * sel, fd[:, None] * sel


def _bcast_mat(ch):
    return ((jnp.arange(HC)[None, :] // ch) == jnp.arange(16)[:, None]).astype(
        jnp.float32)


def kernel(x, edge_index, W1, att_src1, att_dst1, b1, W2, att_src2, att_dst2,
           b2):
    asm1, adm1 = _pack_mats(att_src1, att_dst1, 8)
    asm2, adm2 = _pack_mats(att_src2, att_dst2, HC)
    r16_1 = _bcast_mat(8)
    r16_2 = _bcast_mat(HC)
    npad_e = EPAD - E
    srcp = jnp.concatenate([edge_index[0], jnp.zeros((npad_e,), jnp.int32)])
    dstp = jnp.concatenate([edge_index[1], jnp.full((npad_e,), N, jnp.int32)])
    z64 = jnp.zeros((NPAD, HC), jnp.float32)
    z16 = jnp.zeros((NPAD, 16), jnp.float32)
    b1r = b1.reshape(1, HC)
    b2r = b2.reshape(1, HC)

    h1, as1, ad1, mx1 = _dense1(x, W1, asm1, adm1)
    kv1 = _leaky(mx1[0] + mx1[1]).reshape(1, 16)
    m1, e1 = _edge_kernel_h8(h1, as1, ad1, srcp, dstp, kv1.reshape(16), z64,
                             z16)

    h2, as2, ad2, mx2 = _epi1(m1, e1, h1, as1, ad1, kv1, b1r, r16_1, W2, asm2,
                              adm2)
    kv2 = _leaky(mx2[0] + mx2[1]).reshape(1, 16)
    m2, e2 = _edge_kernel_h1(h2, as2, ad2, srcp, dstp, kv2.reshape(16), z64,
                             z16)

    return _epi2(m2, e2, h2, as2, ad2, kv2, b2r, r16_2)


# speedup vs baseline: 30.9854x; 30.9854x over previous
import jax
import jax.numpy as jnp
from jax import lax
from jax.experimental import pallas as pl
from jax.experimental.pallas import tpu as pltpu
from jax.experimental.pallas import tpu_sc as plsc

N = 10000
F_IN = 256
HC = 64
NPAD = 10112
STRIPE = NPAD // 16
E = 160000
NC, NS = 2, 16
NW = NC * NS
EPW = 5120
EPAD = EPW * NW
CH = 128
NCHUNK = EPW // CH
BM = 400


def _leaky(x):
    return jnp.where(x >= 0, x, 0.2 * x)


def _dense1_body(x_ref, w_ref, asm_ref, adm_ref, h_ref, as_ref, ad_ref, mx_ref):
    i = pl.program_id(0)
    h = jnp.dot(x_ref[...], w_ref[...], preferred_element_type=jnp.float32)
    a_s = jnp.dot(h, asm_ref[...], preferred_element_type=jnp.float32)
    a_d = jnp.dot(h, adm_ref[...], preferred_element_type=jnp.float32)
    h_ref[...] = h
    as_ref[...] = a_s
    ad_ref[...] = a_d

    @pl.when(i == 0)
    def _():
        mx_ref[...] = jnp.full((2, 16), -3.0e38, jnp.float32)

    upd = jnp.concatenate(
        [jnp.max(a_s, axis=0, keepdims=True), jnp.max(a_d, axis=0, keepdims=True)],
        axis=0,
    )
    mx_ref[...] = jnp.maximum(mx_ref[...], upd)


def _dense1(x, w1, asm, adm):
    return pl.pallas_call(
        _dense1_body,
        grid=(N // BM,),
        in_specs=[
            pl.BlockSpec((BM, F_IN), lambda i: (i, 0)),
            pl.BlockSpec((F_IN, HC), lambda i: (0, 0)),
            pl.BlockSpec((HC, 16), lambda i: (0, 0)),
            pl.BlockSpec((HC, 16), lambda i: (0, 0)),
        ],
        out_specs=[
            pl.BlockSpec((BM, HC), lambda i: (i, 0)),
            pl.BlockSpec((BM, 16), lambda i: (i, 0)),
            pl.BlockSpec((BM, 16), lambda i: (i, 0)),
            pl.BlockSpec((2, 16), lambda i: (0, 0)),
        ],
        out_shape=[
            jax.ShapeDtypeStruct((N, HC), jnp.float32),
            jax.ShapeDtypeStruct((N, 16), jnp.float32),
            jax.ShapeDtypeStruct((N, 16), jnp.float32),
            jax.ShapeDtypeStruct((2, 16), jnp.float32),
        ],
    )(x, w1, asm, adm)


def _make_edge_kernel(nheads):
    mesh = plsc.VectorSubcoreMesh(core_axis_name="c", subcore_axis_name="s",
                                  num_cores=NC, num_subcores=NS)

    def body(h_hbm, as_hbm, ad_hbm, src_hbm, dst_hbm, kv_hbm, z64_hbm, z16_hbm,
             m_out, e_out, sidx, didx, hrows, arows, drows, kv_v, ebuf,
             acc_m, acc_e, sem1, sem2, sem3):
        c = lax.axis_index("c")
        s = lax.axis_index("s")
        wid = s * NC + c
        pltpu.sync_copy(z64_hbm.at[pl.ds(s * STRIPE, STRIPE)],
                        acc_m.at[pl.ds(s * STRIPE, STRIPE)])
        pltpu.sync_copy(z16_hbm.at[pl.ds(s * STRIPE, STRIPE)],
                        acc_e.at[pl.ds(s * STRIPE, STRIPE)])
        pltpu.sync_copy(kv_hbm, kv_v)
        plsc.subcore_barrier()

        kv = kv_v[...]
        iota = lax.iota(jnp.int32, 16)
        half = lax.shift_right_logical(iota, 3)

        def chunk_body(ch, carry):
            base = wid * EPW + ch * CH
            pltpu.sync_copy(src_hbm.at[pl.ds(base, CH)], sidx)
            pltpu.sync_copy(dst_hbm.at[pl.ds(base, CH)], didx)
            cp1 = pltpu.async_copy(h_hbm.at[sidx], hrows, sem1)
            cp2 = pltpu.async_copy(as_hbm.at[sidx], arows, sem2)
            cp3 = pltpu.async_copy(ad_hbm.at[didx], drows, sem3)
            cp1.wait()
            cp2.wait()
            cp3.wait()

            def edge_body(i, carry2):
                a = arows[i, :] + drows[i, :]
                e = jnp.exp(_leaky(a) - kv)
                if nheads == 8:
                    ebuf[...] = e
                arows[i, :] = e
                for j in range(4):
                    if nheads == 8:
                        eb = plsc.load_gather(ebuf, [half + (2 * j)])
                    else:
                        eb = e
                    hrows[i, pl.ds(16 * j, 16)] = hrows[i, pl.ds(16 * j, 16)] * eb
                return carry2

            lax.fori_loop(0, CH, edge_body, 0)
            pltpu.sync_copy(hrows, acc_m.at[didx], add=True)
            pltpu.sync_copy(arows, acc_e.at[didx], add=True)
            return carry

        lax.fori_loop(0, NCHUNK, chunk_body, 0)
        plsc.subcore_barrier()
        pltpu.sync_copy(acc_m.at[pl.ds(s * STRIPE, STRIPE)],
                        m_out.at[c, pl.ds(s * STRIPE, STRIPE)])
        pltpu.sync_copy(acc_e.at[pl.ds(s * STRIPE, STRIPE)],
                        e_out.at[c, pl.ds(s * STRIPE, STRIPE)])

    return pl.kernel(
        body,
        out_type=[
            jax.ShapeDtypeStruct((NC, NPAD, HC), jnp.float32),
            jax.ShapeDtypeStruct((NC, NPAD, 16), jnp.float32),
        ],
        mesh=mesh,
        compiler_params=pltpu.CompilerParams(needs_layout_passes=False,
                                             use_tc_tiling_on_sc=False),
        scratch_types=[
            pltpu.VMEM((CH,), jnp.int32),
            pltpu.VMEM((CH,), jnp.int32),
            pltpu.VMEM((CH, HC), jnp.float32),
            pltpu.VMEM((CH, 16), jnp.float32),
            pltpu.VMEM((CH, 16), jnp.float32),
            pltpu.VMEM((16,), jnp.float32),
            pltpu.VMEM((16,), jnp.float32),
            pltpu.VMEM_SHARED((NPAD, HC), jnp.float32),
            pltpu.VMEM_SHARED((NPAD, 16), jnp.float32),
            pltpu.SemaphoreType.DMA,
            pltpu.SemaphoreType.DMA,
            pltpu.SemaphoreType.DMA,
        ],
    )


import functools


@functools.lru_cache(maxsize=2)
def _get_edge_kernel(nheads):
    return _make_edge_kernel(nheads)


def _epi1_body(m_ref, e_ref, h_ref, as_ref, ad_ref, kv_ref, b_ref, r_ref,
               w2_ref, asm_ref, adm_ref, h2_ref, as2_ref, ad2_ref, mx_ref):
    i = pl.program_id(0)
    m = m_ref[...][0] + m_ref[...][1]
    e2 = e_ref[...][0] + e_ref[...][1]
    a = as_ref[...] + ad_ref[...]
    es = jnp.exp(_leaky(a) - kv_ref[...])
    den = jnp.dot(e2 + es, r_ref[...], preferred_element_type=jnp.float32)
    esb = jnp.dot(es, r_ref[...], preferred_element_type=jnp.float32)
    num = m + h_ref[...] * esb
    h1 = num / (den + 1e-16) + b_ref[...]
    h1e = jnp.where(h1 > 0, h1, jnp.exp(h1) - 1.0)
    h2 = jnp.dot(h1e, w2_ref[...], preferred_element_type=jnp.float32)
    a_s2 = jnp.dot(h2, asm_ref[...], preferred_element_type=jnp.float32)
    a_d2 = jnp.dot(h2, adm_ref[...], preferred_element_type=jnp.float32)
    h2_ref[...] = h2
    as2_ref[...] = a_s2
    ad2_ref[...] = a_d2

    @pl.when(i == 0)
    def _():
        mx_ref[...] = jnp.full((2, 16), -3.0e38, jnp.float32)

    upd = jnp.concatenate(
        [jnp.max(a_s2, axis=0, keepdims=True), jnp.max(a_d2, axis=0, keepdims=True)],
        axis=0,
    )
    mx_ref[...] = jnp.maximum(mx_ref[...], upd)


def _epi1(m1, e1, h1, as1, ad1, kv1, b1, r16, w2, asm2, adm2):
    return pl.pallas_call(
        _epi1_body,
        grid=(N // BM,),
        in_specs=[
            pl.BlockSpec((2, BM, HC), lambda i: (0, i, 0)),
            pl.BlockSpec((2, BM, 16), lambda i: (0, i, 0)),
            pl.BlockSpec((BM, HC), lambda i: (i, 0)),
            pl.BlockSpec((BM, 16), lambda i: (i, 0)),
            pl.BlockSpec((BM, 16), lambda i: (i, 0)),
            pl.BlockSpec((1, 16), lambda i: (0, 0)),
            pl.BlockSpec((1, HC), lambda i: (0, 0)),
            pl.BlockSpec((16, HC), lambda i: (0, 0)),
            pl.BlockSpec((HC, HC), lambda i: (0, 0)),
            pl.BlockSpec((HC, 16), lambda i: (0, 0)),
            pl.BlockSpec((HC, 16), lambda i: (0, 0)),
        ],
        out_specs=[
            pl.BlockSpec((BM, HC), lambda i: (i, 0)),
            pl.BlockSpec((BM, 16), lambda i: (i, 0)),
            pl.BlockSpec((BM, 16), lambda i: (i, 0)),
            pl.BlockSpec((2, 16), lambda i: (0, 0)),
        ],
        out_shape=[
            jax.ShapeDtypeStruct((N, HC), jnp.float32),
            jax.ShapeDtypeStruct((N, 16), jnp.float32),
            jax.ShapeDtypeStruct((N, 16), jnp.float32),
            jax.ShapeDtypeStruct((2, 16), jnp.float32),
        ],
    )(m1, e1, h1, as1, ad1, kv1, b1, r16, w2, asm2, adm2)


def _epi2_body(m_ref, e_ref, h_ref, as_ref, ad_ref, kv_ref, b_ref, r_ref,
               out_ref):
    m = m_ref[...][0] + m_ref[...][1]
    e2 = e_ref[...][0] + e_ref[...][1]
    a = as_ref[...] + ad_ref[...]
    es = jnp.exp(_leaky(a) - kv_ref[...])
    den = jnp.dot(e2 + es, r_ref[...], preferred_element_type=jnp.float32)
    esb = jnp.dot(es, r_ref[...], preferred_element_type=jnp.float32)
    num = m + h_ref[...] * esb
    o = num / (den + 1e-16) + b_ref[...]
    mx = jnp.max(o, axis=1, keepdims=True)
    z = o - mx
    lse = jnp.log(jnp.sum(jnp.exp(z), axis=1, keepdims=True))
    out_ref[...] = z - lse


def _epi2(m2, e2, h2, as2, ad2, kv2, b2, r16):
    return pl.pallas_call(
        _epi2_body,
        grid=(N // BM,),
        in_specs=[
            pl.BlockSpec((2, BM, HC), lambda i: (0, i, 0)),
            pl.BlockSpec((2, BM, 16), lambda i: (0, i, 0)),
            pl.BlockSpec((BM, HC), lambda i: (i, 0)),
            pl.BlockSpec((BM, 16), lambda i: (i, 0)),
            pl.BlockSpec((BM, 16), lambda i: (i, 0)),
            pl.BlockSpec((1, 16), lambda i: (0, 0)),
            pl.BlockSpec((1, HC), lambda i: (0, 0)),
            pl.BlockSpec((16, HC), lambda i: (0, 0)),
        ],
        out_specs=pl.BlockSpec((BM, HC), lambda i: (i, 0)),
        out_shape=jax.ShapeDtypeStruct((N, HC), jnp.float32),
    )(m2, e2, h2, as2, ad2, kv2, b2, r16)


def _pack_mats(att_src, att_dst, ch):
    fs = att_src.reshape(HC)
    fd = att_dst.reshape(HC)
    rows = jnp.arange(HC)
    cols = jnp.arange(16)
    if ch == HC:
        sel = jnp.ones((HC, 16), jnp.float32)
    else:
        sel = (cols[None, :] == (rows[:, None] // ch)).astype(jnp.float32)
    return fs[:, None] * sel, fd[:, None] * sel


def _bcast_mat(ch):
    return ((jnp.arange(HC)[None, :] // ch) == jnp.arange(16)[:, None]).astype(
        jnp.float32)


def kernel(x, edge_index, W1, att_src1, att_dst1, b1, W2, att_src2, att_dst2,
           b2):
    asm1, adm1 = _pack_mats(att_src1, att_dst1, 8)
    asm2, adm2 = _pack_mats(att_src2, att_dst2, HC)
    r16_1 = _bcast_mat(8)
    r16_2 = _bcast_mat(HC)
    npad_e = EPAD - E
    srcp = jnp.concatenate([edge_index[0], jnp.zeros((npad_e,), jnp.int32)])
    dstp = jnp.concatenate([edge_index[1], jnp.full((npad_e,), N, jnp.int32)])
    z64 = jnp.zeros((NPAD, HC), jnp.float32)
    z16 = jnp.zeros((NPAD, 16), jnp.float32)
    b1r = b1.reshape(1, HC)
    b2r = b2.reshape(1, HC)

    h1, as1, ad1, mx1 = _dense1(x, W1, asm1, adm1)
    kv1 = _leaky(mx1[0] + mx1[1]).reshape(1, 16)
    m1, e1 = _get_edge_kernel(8)(h1, as1, ad1, srcp, dstp, kv1.reshape(16),
                                 z64, z16)

    h2, as2, ad2, mx2 = _epi1(m1, e1, h1, as1, ad1, kv1, b1r, r16_1, W2, asm2,
                              adm2)
    kv2 = _leaky(mx2[0] + mx2[1]).reshape(1, 16)
    m2, e2 = _get_edge_kernel(1)(h2, as2, ad2, srcp, dstp, kv2.reshape(16),
                                 z64, z16)

    return _epi2(m2, e2, h2, as2, ad2, kv2, b2r, r16_2)


# speedup vs baseline: 43.0613x; 1.3897x over previous
import jax
import jax.numpy as jnp
from jax import lax
from jax.experimental import pallas as pl
from jax.experimental.pallas import tpu as pltpu
from jax.experimental.pallas import tpu_sc as plsc

N = 10000
F_IN = 256
HC = 64
NPAD = 10112
STRIPE = NPAD // 16
E = 160000
NC, NS = 2, 16
NW = NC * NS
EPW = 5120
EPAD = EPW * NW
CH = 128
NCHUNK = EPW // CH
BM = 400


def _leaky(x):
    return jnp.where(x >= 0, x, 0.2 * x)


def _dense1_body(x_ref, w_ref, asm_ref, adm_ref, h_ref, as_ref, ad_ref, mx_ref):
    i = pl.program_id(0)
    h = jnp.dot(x_ref[...], w_ref[...], preferred_element_type=jnp.float32)
    a_s = jnp.dot(h, asm_ref[...], preferred_element_type=jnp.float32)
    a_d = jnp.dot(h, adm_ref[...], preferred_element_type=jnp.float32)
    h_ref[...] = h
    as_ref[...] = a_s
    ad_ref[...] = a_d

    @pl.when(i == 0)
    def _():
        mx_ref[...] = jnp.full((2, 16), -3.0e38, jnp.float32)

    upd = jnp.concatenate(
        [jnp.max(a_s, axis=0, keepdims=True), jnp.max(a_d, axis=0, keepdims=True)],
        axis=0,
    )
    mx_ref[...] = jnp.maximum(mx_ref[...], upd)


def _dense1(x, w1, asm, adm):
    return pl.pallas_call(
        _dense1_body,
        grid=(N // BM,),
        in_specs=[
            pl.BlockSpec((BM, F_IN), lambda i: (i, 0)),
            pl.BlockSpec((F_IN, HC), lambda i: (0, 0)),
            pl.BlockSpec((HC, 16), lambda i: (0, 0)),
            pl.BlockSpec((HC, 16), lambda i: (0, 0)),
        ],
        out_specs=[
            pl.BlockSpec((BM, HC), lambda i: (i, 0)),
            pl.BlockSpec((BM, 16), lambda i: (i, 0)),
            pl.BlockSpec((BM, 16), lambda i: (i, 0)),
            pl.BlockSpec((2, 16), lambda i: (0, 0)),
        ],
        out_shape=[
            jax.ShapeDtypeStruct((N, HC), jnp.float32),
            jax.ShapeDtypeStruct((N, 16), jnp.float32),
            jax.ShapeDtypeStruct((N, 16), jnp.float32),
            jax.ShapeDtypeStruct((2, 16), jnp.float32),
        ],
    )(x, w1, asm, adm)


def _make_edge_kernel(nheads):
    mesh = plsc.VectorSubcoreMesh(core_axis_name="c", subcore_axis_name="s",
                                  num_cores=NC, num_subcores=NS)

    def body(h_hbm, as_hbm, ad_hbm, src_hbm, dst_hbm, kv_hbm, z64_hbm, z16_hbm,
             m_out, e_out, sidx, didx, hrows, arows, drows, kv_v,
             acc_m, acc_e, sem1, sem2, sem3):
        c = lax.axis_index("c")
        s = lax.axis_index("s")
        wid = s * NC + c
        pltpu.sync_copy(z64_hbm.at[pl.ds(s * STRIPE, STRIPE)],
                        acc_m.at[pl.ds(s * STRIPE, STRIPE)])
        pltpu.sync_copy(z16_hbm.at[pl.ds(s * STRIPE, STRIPE)],
                        acc_e.at[pl.ds(s * STRIPE, STRIPE)])
        pltpu.sync_copy(kv_hbm, kv_v)
        plsc.subcore_barrier()

        kv = kv_v[...]
        iota = lax.iota(jnp.int32, 16)
        half = lax.shift_right_logical(iota, 3)

        def chunk_body(ch, carry):
            base = wid * EPW + ch * CH
            pltpu.sync_copy(src_hbm.at[pl.ds(base, CH)], sidx)
            pltpu.sync_copy(dst_hbm.at[pl.ds(base, CH)], didx)
            cp1 = pltpu.async_copy(h_hbm.at[sidx], hrows, sem1)
            cp2 = pltpu.async_copy(as_hbm.at[sidx], arows, sem2)
            cp3 = pltpu.async_copy(ad_hbm.at[didx], drows, sem3)
            cp1.wait()
            cp2.wait()
            cp3.wait()

            @plsc.parallel_loop(0, CH, unroll=8)
            def edge_body(i):
                a = arows[i, :] + drows[i, :]
                e = jnp.exp(_leaky(a) - kv)
                arows[i, :] = e
                if nheads == 8:
                    fi = jnp.full((16,), i, jnp.int32)
                    for j in range(4):
                        eb = plsc.load_gather(arows, [fi, half + (2 * j)])
                        hrows[i, pl.ds(16 * j, 16)] = (
                            hrows[i, pl.ds(16 * j, 16)] * eb)
                else:
                    for j in range(4):
                        hrows[i, pl.ds(16 * j, 16)] = (
                            hrows[i, pl.ds(16 * j, 16)] * e)
            pltpu.sync_copy(hrows, acc_m.at[didx], add=True)
            pltpu.sync_copy(arows, acc_e.at[didx], add=True)
            return carry

        lax.fori_loop(0, NCHUNK, chunk_body, 0)
        plsc.subcore_barrier()
        pltpu.sync_copy(acc_m.at[pl.ds(s * STRIPE, STRIPE)],
                        m_out.at[c, pl.ds(s * STRIPE, STRIPE)])
        pltpu.sync_copy(acc_e.at[pl.ds(s * STRIPE, STRIPE)],
                        e_out.at[c, pl.ds(s * STRIPE, STRIPE)])

    return pl.kernel(
        body,
        out_type=[
            jax.ShapeDtypeStruct((NC, NPAD, HC), jnp.float32),
            jax.ShapeDtypeStruct((NC, NPAD, 16), jnp.float32),
        ],
        mesh=mesh,
        compiler_params=pltpu.CompilerParams(needs_layout_passes=False,
                                             use_tc_tiling_on_sc=False),
        scratch_types=[
            pltpu.VMEM((CH,), jnp.int32),
            pltpu.VMEM((CH,), jnp.int32),
            pltpu.VMEM((CH, HC), jnp.float32),
            pltpu.VMEM((CH, 16), jnp.float32),
            pltpu.VMEM((CH, 16), jnp.float32),
            pltpu.VMEM((16,), jnp.float32),
            pltpu.VMEM_SHARED((NPAD, HC), jnp.float32),
            pltpu.VMEM_SHARED((NPAD, 16), jnp.float32),
            pltpu.SemaphoreType.DMA,
            pltpu.SemaphoreType.DMA,
            pltpu.SemaphoreType.DMA,
        ],
    )


import functools


@functools.lru_cache(maxsize=2)
def _get_edge_kernel(nheads):
    return _make_edge_kernel(nheads)


def _epi1_body(m_ref, e_ref, h_ref, as_ref, ad_ref, kv_ref, b_ref, r_ref,
               w2_ref, asm_ref, adm_ref, h2_ref, as2_ref, ad2_ref, mx_ref):
    i = pl.program_id(0)
    m = m_ref[...][0] + m_ref[...][1]
    e2 = e_ref[...][0] + e_ref[...][1]
    a = as_ref[...] + ad_ref[...]
    es = jnp.exp(_leaky(a) - kv_ref[...])
    den = jnp.dot(e2 + es, r_ref[...], preferred_element_type=jnp.float32)
    esb = jnp.dot(es, r_ref[...], preferred_element_type=jnp.float32)
    num = m + h_ref[...] * esb
    h1 = num / (den + 1e-16) + b_ref[...]
    h1e = jnp.where(h1 > 0, h1, jnp.exp(h1) - 1.0)
    h2 = jnp.dot(h1e, w2_ref[...], preferred_element_type=jnp.float32)
    a_s2 = jnp.dot(h2, asm_ref[...], preferred_element_type=jnp.float32)
    a_d2 = jnp.dot(h2, adm_ref[...], preferred_element_type=jnp.float32)
    h2_ref[...] = h2
    as2_ref[...] = a_s2
    ad2_ref[...] = a_d2

    @pl.when(i == 0)
    def _():
        mx_ref[...] = jnp.full((2, 16), -3.0e38, jnp.float32)

    upd = jnp.concatenate(
        [jnp.max(a_s2, axis=0, keepdims=True), jnp.max(a_d2, axis=0, keepdims=True)],
        axis=0,
    )
    mx_ref[...] = jnp.maximum(mx_ref[...], upd)


def _epi1(m1, e1, h1, as1, ad1, kv1, b1, r16, w2, asm2, adm2):
    return pl.pallas_call(
        _epi1_body,
        grid=(N // BM,),
        in_specs=[
            pl.BlockSpec((2, BM, HC), lambda i: (0, i, 0)),
            pl.BlockSpec((2, BM, 16), lambda i: (0, i, 0)),
            pl.BlockSpec((BM, HC), lambda i: (i, 0)),
            pl.BlockSpec((BM, 16), lambda i: (i, 0)),
            pl.BlockSpec((BM, 16), lambda i: (i, 0)),
            pl.BlockSpec((1, 16), lambda i: (0, 0)),
            pl.BlockSpec((1, HC), lambda i: (0, 0)),
            pl.BlockSpec((16, HC), lambda i: (0, 0)),
            pl.BlockSpec((HC, HC), lambda i: (0, 0)),
            pl.BlockSpec((HC, 16), lambda i: (0, 0)),
            pl.BlockSpec((HC, 16), lambda i: (0, 0)),
        ],
        out_specs=[
            pl.BlockSpec((BM, HC), lambda i: (i, 0)),
            pl.BlockSpec((BM, 16), lambda i: (i, 0)),
            pl.BlockSpec((BM, 16), lambda i: (i, 0)),
            pl.BlockSpec((2, 16), lambda i: (0, 0)),
        ],
        out_shape=[
            jax.ShapeDtypeStruct((N, HC), jnp.float32),
            jax.ShapeDtypeStruct((N, 16), jnp.float32),
            jax.ShapeDtypeStruct((N, 16), jnp.float32),
            jax.ShapeDtypeStruct((2, 16), jnp.float32),
        ],
    )(m1, e1, h1, as1, ad1, kv1, b1, r16, w2, asm2, adm2)


def _epi2_body(m_ref, e_ref, h_ref, as_ref, ad_ref, kv_ref, b_ref, r_ref,
               out_ref):
    m = m_ref[...][0] + m_ref[...][1]
    e2 = e_ref[...][0] + e_ref[...][1]
    a = as_ref[...] + ad_ref[...]
    es = jnp.exp(_leaky(a) - kv_ref[...])
    den = jnp.dot(e2 + es, r_ref[...], preferred_element_type=jnp.float32)
    esb = jnp.dot(es, r_ref[...], preferred_element_type=jnp.float32)
    num = m + h_ref[...] * esb
    o = num / (den + 1e-16) + b_ref[...]
    mx = jnp.max(o, axis=1, keepdims=True)
    z = o - mx
    lse = jnp.log(jnp.sum(jnp.exp(z), axis=1, keepdims=True))
    out_ref[...] = z - lse


def _epi2(m2, e2, h2, as2, ad2, kv2, b2, r16):
    return pl.pallas_call(
        _epi2_body,
        grid=(N // BM,),
        in_specs=[
            pl.BlockSpec((2, BM, HC), lambda i: (0, i, 0)),
            pl.BlockSpec((2, BM, 16), lambda i: (0, i, 0)),
            pl.BlockSpec((BM, HC), lambda i: (i, 0)),
            pl.BlockSpec((BM, 16), lambda i: (i, 0)),
            pl.BlockSpec((BM, 16), lambda i: (i, 0)),
            pl.BlockSpec((1, 16), lambda i: (0, 0)),
            pl.BlockSpec((1, HC), lambda i: (0, 0)),
            pl.BlockSpec((16, HC), lambda i: (0, 0)),
        ],
        out_specs=pl.BlockSpec((BM, HC), lambda i: (i, 0)),
        out_shape=jax.ShapeDtypeStruct((N, HC), jnp.float32),
    )(m2, e2, h2, as2, ad2, kv2, b2, r16)


def _pack_mats(att_src, att_dst, ch):
    fs = att_src.reshape(HC)
    fd = att_dst.reshape(HC)
    rows = jnp.arange(HC)
    cols = jnp.arange(16)
    if ch == HC:
        sel = jnp.ones((HC, 16), jnp.float32)
    else:
        sel = (cols[None, :] == (rows[:, None] // ch)).astype(jnp.float32)
    return fs[:, None] * sel, fd[:, None] * sel


def _bcast_mat(ch):
    return ((jnp.arange(HC)[None, :] // ch) == jnp.arange(16)[:, None]).astype(
        jnp.float32)


def kernel(x, edge_index, W1, att_src1, att_dst1, b1, W2, att_src2, att_dst2,
           b2):
    asm1, adm1 = _pack_mats(att_src1, att_dst1, 8)
    asm2, adm2 = _pack_mats(att_src2, att_dst2, HC)
    r16_1 = _bcast_mat(8)
    r16_2 = _bcast_mat(HC)
    npad_e = EPAD - E
    srcp = jnp.concatenate([edge_index[0], jnp.zeros((npad_e,), jnp.int32)])
    dstp = jnp.concatenate([edge_index[1], jnp.full((npad_e,), N, jnp.int32)])
    z64 = jnp.zeros((NPAD, HC), jnp.float32)
    z16 = jnp.zeros((NPAD, 16), jnp.float32)
    b1r = b1.reshape(1, HC)
    b2r = b2.reshape(1, HC)

    h1, as1, ad1, mx1 = _dense1(x, W1, asm1, adm1)
    kv1 = _leaky(mx1[0] + mx1[1]).reshape(1, 16)
    m1, e1 = _get_edge_kernel(8)(h1, as1, ad1, srcp, dstp, kv1.reshape(16),
                                 z64, z16)

    h2, as2, ad2, mx2 = _epi1(m1, e1, h1, as1, ad1, kv1, b1r, r16_1, W2, asm2,
                              adm2)
    kv2 = _leaky(mx2[0] + mx2[1]).reshape(1, 16)
    m2, e2 = _get_edge_kernel(1)(h2, as2, ad2, srcp, dstp, kv2.reshape(16),
                                 z64, z16)

    return _epi2(m2, e2, h2, as2, ad2, kv2, b2r, r16_2)


# speedup vs baseline: 62.4772x; 1.4509x over previous
import jax
import jax.numpy as jnp
from jax import lax
from jax.experimental import pallas as pl
from jax.experimental.pallas import tpu as pltpu
from jax.experimental.pallas import tpu_sc as plsc

N = 10000
F_IN = 256
HC = 64
NPAD = 10112
STRIPE = NPAD // 16
E = 160000
NC, NS = 2, 16
NW = NC * NS
EPW = 5120
EPAD = EPW * NW
CH = 128
NCHUNK = EPW // CH
BM = 400


def _leaky(x):
    return jnp.where(x >= 0, x, 0.2 * x)


def _dense1_body(x_ref, w_ref, asm_ref, adm_ref, h_ref, as_ref, ad_ref, mx_ref):
    i = pl.program_id(0)
    h = jnp.dot(x_ref[...], w_ref[...], preferred_element_type=jnp.float32)
    a_s = jnp.dot(h, asm_ref[...], preferred_element_type=jnp.float32)
    a_d = jnp.dot(h, adm_ref[...], preferred_element_type=jnp.float32)
    h_ref[...] = h
    as_ref[...] = a_s
    ad_ref[...] = a_d

    @pl.when(i == 0)
    def _():
        mx_ref[...] = jnp.full((2, 16), -3.0e38, jnp.float32)

    upd = jnp.concatenate(
        [jnp.max(a_s, axis=0, keepdims=True), jnp.max(a_d, axis=0, keepdims=True)],
        axis=0,
    )
    mx_ref[...] = jnp.maximum(mx_ref[...], upd)


def _dense1(x, w1, asm, adm):
    return pl.pallas_call(
        _dense1_body,
        grid=(N // BM,),
        in_specs=[
            pl.BlockSpec((BM, F_IN), lambda i: (i, 0)),
            pl.BlockSpec((F_IN, HC), lambda i: (0, 0)),
            pl.BlockSpec((HC, 16), lambda i: (0, 0)),
            pl.BlockSpec((HC, 16), lambda i: (0, 0)),
        ],
        out_specs=[
            pl.BlockSpec((BM, HC), lambda i: (i, 0)),
            pl.BlockSpec((BM, 16), lambda i: (i, 0)),
            pl.BlockSpec((BM, 16), lambda i: (i, 0)),
            pl.BlockSpec((2, 16), lambda i: (0, 0)),
        ],
        out_shape=[
            jax.ShapeDtypeStruct((N, HC), jnp.float32),
            jax.ShapeDtypeStruct((N, 16), jnp.float32),
            jax.ShapeDtypeStruct((N, 16), jnp.float32),
            jax.ShapeDtypeStruct((2, 16), jnp.float32),
        ],
    )(x, w1, asm, adm)


def _make_edge_kernel(nheads):
    mesh = plsc.VectorSubcoreMesh(core_axis_name="c", subcore_axis_name="s",
                                  num_cores=NC, num_subcores=NS)

    def body(h_hbm, as_hbm, ad_hbm, src_hbm, dst_hbm, kv_hbm, z64_hbm, z16_hbm,
             m_out, e_out, sidx, didx, sdix, hrows, arows, drows, orows,
             erows, kv_v, acc_m, acc_e, semg, sems):
        c = lax.axis_index("c")
        s = lax.axis_index("s")
        wid = s * NC + c
        pltpu.sync_copy(z64_hbm.at[pl.ds(s * STRIPE, STRIPE)],
                        acc_m.at[pl.ds(s * STRIPE, STRIPE)])
        pltpu.sync_copy(z16_hbm.at[pl.ds(s * STRIPE, STRIPE)],
                        acc_e.at[pl.ds(s * STRIPE, STRIPE)])
        pltpu.sync_copy(kv_hbm, kv_v)
        plsc.subcore_barrier()

        kv = kv_v[...]
        iota = lax.iota(jnp.int32, 16)
        half = lax.shift_right_logical(iota, 3)

        def issue_gathers(k, b):
            base = wid * EPW + k * CH
            pltpu.sync_copy(src_hbm.at[pl.ds(base, CH)], sidx.at[b])
            pltpu.sync_copy(dst_hbm.at[pl.ds(base, CH)], didx.at[b])
            pltpu.async_copy(h_hbm.at[sidx.at[b]], hrows.at[b], semg.at[b])
            pltpu.async_copy(as_hbm.at[sidx.at[b]], arows.at[b], semg.at[b])
            pltpu.async_copy(ad_hbm.at[didx.at[b]], drows.at[b], semg.at[b])

        def wait_gathers(b):
            pltpu.make_async_copy(h_hbm.at[sidx.at[b]], hrows.at[b],
                                  semg.at[b]).wait()
            pltpu.make_async_copy(as_hbm.at[sidx.at[b]], arows.at[b],
                                  semg.at[b]).wait()
            pltpu.make_async_copy(ad_hbm.at[didx.at[b]], drows.at[b],
                                  semg.at[b]).wait()

        def compute(b):
            hr, ar, dr = hrows.at[b], arows.at[b], drows.at[b]
            orr, er = orows.at[b], erows.at[b]

            @plsc.parallel_loop(0, CH, unroll=8)
            def edge_body(i):
                a = ar[i, :] + dr[i, :]
                e = jnp.exp(_leaky(a) - kv)
                er[i, :] = e
                if nheads == 8:
                    fi = jnp.full((16,), i, jnp.int32)
                    for j in range(4):
                        eb = plsc.load_gather(er, [fi, half + (2 * j)])
                        orr[i, pl.ds(16 * j, 16)] = (
                            hr[i, pl.ds(16 * j, 16)] * eb)
                else:
                    for j in range(4):
                        orr[i, pl.ds(16 * j, 16)] = (
                            hr[i, pl.ds(16 * j, 16)] * e)

        def issue_scatters(b):
            pltpu.async_copy(orows.at[b], acc_m.at[sdix.at[b]], sems.at[b],
                             add=True)
            pltpu.async_copy(erows.at[b], acc_e.at[sdix.at[b]], sems.at[b],
                             add=True)

        def wait_scatters(b):
            pltpu.make_async_copy(orows.at[b], acc_m.at[sdix.at[b]],
                                  sems.at[b]).wait()
            pltpu.make_async_copy(erows.at[b], acc_e.at[sdix.at[b]],
                                  sems.at[b]).wait()

        issue_gathers(0, 0)

        def pair_body(k2, carry):
            k = 2 * k2
            issue_gathers(k + 1, 1)
            wait_gathers(0)

            @pl.when(k2 > 0)
            def _():
                wait_scatters(0)

            pltpu.sync_copy(dst_hbm.at[pl.ds(wid * EPW + k * CH, CH)],
                            sdix.at[0])
            compute(0)
            issue_scatters(0)

            @pl.when(k2 < NCHUNK // 2 - 1)
            def _():
                issue_gathers(k + 2, 0)

            wait_gathers(1)

            @pl.when(k2 > 0)
            def _():
                wait_scatters(1)

            pltpu.sync_copy(dst_hbm.at[pl.ds(wid * EPW + (k + 1) * CH, CH)],
                            sdix.at[1])
            compute(1)
            issue_scatters(1)
            return carry

        lax.fori_loop(0, NCHUNK // 2, pair_body, 0)
        wait_scatters(0)
        wait_scatters(1)
        plsc.subcore_barrier()
        pltpu.sync_copy(acc_m.at[pl.ds(s * STRIPE, STRIPE)],
                        m_out.at[c, pl.ds(s * STRIPE, STRIPE)])
        pltpu.sync_copy(acc_e.at[pl.ds(s * STRIPE, STRIPE)],
                        e_out.at[c, pl.ds(s * STRIPE, STRIPE)])

    return pl.kernel(
        body,
        out_type=[
            jax.ShapeDtypeStruct((NC, NPAD, HC), jnp.float32),
            jax.ShapeDtypeStruct((NC, NPAD, 16), jnp.float32),
        ],
        mesh=mesh,
        compiler_params=pltpu.CompilerParams(needs_layout_passes=False,
                                             use_tc_tiling_on_sc=False),
        scratch_types=[
            pltpu.VMEM((2, CH), jnp.int32),
            pltpu.VMEM((2, CH), jnp.int32),
            pltpu.VMEM((2, CH), jnp.int32),
            pltpu.VMEM((2, CH, HC), jnp.float32),
            pltpu.VMEM((2, CH, 16), jnp.float32),
            pltpu.VMEM((2, CH, 16), jnp.float32),
            pltpu.VMEM((2, CH, HC), jnp.float32),
            pltpu.VMEM((2, CH, 16), jnp.float32),
            pltpu.VMEM((16,), jnp.float32),
            pltpu.VMEM_SHARED((NPAD, HC), jnp.float32),
            pltpu.VMEM_SHARED((NPAD, 16), jnp.float32),
            pltpu.SemaphoreType.DMA((2,)),
            pltpu.SemaphoreType.DMA((2,)),
        ],
    )


import functools


@functools.lru_cache(maxsize=2)
def _get_edge_kernel(nheads):
    return _make_edge_kernel(nheads)


def _epi1_body(m_ref, e_ref, h_ref, as_ref, ad_ref, kv_ref, b_ref, r_ref,
               w2_ref, asm_ref, adm_ref, h2_ref, as2_ref, ad2_ref, mx_ref):
    i = pl.program_id(0)
    m = m_ref[...][0] + m_ref[...][1]
    e2 = e_ref[...][0] + e_ref[...][1]
    a = as_ref[...] + ad_ref[...]
    es = jnp.exp(_leaky(a) - kv_ref[...])
    den = jnp.dot(e2 + es, r_ref[...], preferred_element_type=jnp.float32)
    esb = jnp.dot(es, r_ref[...], preferred_element_type=jnp.float32)
    num = m + h_ref[...] * esb
    h1 = num / (den + 1e-16) + b_ref[...]
    h1e = jnp.where(h1 > 0, h1, jnp.exp(h1) - 1.0)
    h2 = jnp.dot(h1e, w2_ref[...], preferred_element_type=jnp.float32)
    a_s2 = jnp.dot(h2, asm_ref[...], preferred_element_type=jnp.float32)
    a_d2 = jnp.dot(h2, adm_ref[...], preferred_element_type=jnp.float32)
    h2_ref[...] = h2
    as2_ref[...] = a_s2
    ad2_ref[...] = a_d2

    @pl.when(i == 0)
    def _():
        mx_ref[...] = jnp.full((2, 16), -3.0e38, jnp.float32)

    upd = jnp.concatenate(
        [jnp.max(a_s2, axis=0, keepdims=True), jnp.max(a_d2, axis=0, keepdims=True)],
        axis=0,
    )
    mx_ref[...] = jnp.maximum(mx_ref[...], upd)


def _epi1(m1, e1, h1, as1, ad1, kv1, b1, r16, w2, asm2, adm2):
    return pl.pallas_call(
        _epi1_body,
        grid=(N // BM,),
        in_specs=[
            pl.BlockSpec((2, BM, HC), lambda i: (0, i, 0)),
            pl.BlockSpec((2, BM, 16), lambda i: (0, i, 0)),
            pl.BlockSpec((BM, HC), lambda i: (i, 0)),
            pl.BlockSpec((BM, 16), lambda i: (i, 0)),
            pl.BlockSpec((BM, 16), lambda i: (i, 0)),
            pl.BlockSpec((1, 16), lambda i: (0, 0)),
            pl.BlockSpec((1, HC), lambda i: (0, 0)),
            pl.BlockSpec((16, HC), lambda i: (0, 0)),
            pl.BlockSpec((HC, HC), lambda i: (0, 0)),
            pl.BlockSpec((HC, 16), lambda i: (0, 0)),
            pl.BlockSpec((HC, 16), lambda i: (0, 0)),
        ],
        out_specs=[
            pl.BlockSpec((BM, HC), lambda i: (i, 0)),
            pl.BlockSpec((BM, 16), lambda i: (i, 0)),
            pl.BlockSpec((BM, 16), lambda i: (i, 0)),
            pl.BlockSpec((2, 16), lambda i: (0, 0)),
        ],
        out_shape=[
            jax.ShapeDtypeStruct((N, HC), jnp.float32),
            jax.ShapeDtypeStruct((N, 16), jnp.float32),
            jax.ShapeDtypeStruct((N, 16), jnp.float32),
            jax.ShapeDtypeStruct((2, 16), jnp.float32),
        ],
    )(m1, e1, h1, as1, ad1, kv1, b1, r16, w2, asm2, adm2)


def _epi2_body(m_ref, e_ref, h_ref, as_ref, ad_ref, kv_ref, b_ref, r_ref,
               out_ref):
    m = m_ref[...][0] + m_ref[...][1]
    e2 = e_ref[...][0] + e_ref[...][1]
    a = as_ref[...] + ad_ref[...]
    es = jnp.exp(_leaky(a) - kv_ref[...])
    den = jnp.dot(e2 + es, r_ref[...], preferred_element_type=jnp.float32)
    esb = jnp.dot(es, r_ref[...], preferred_element_type=jnp.float32)
    num = m + h_ref[...] * esb
    o = num / (den + 1e-16) + b_ref[...]
    mx = jnp.max(o, axis=1, keepdims=True)
    z = o - mx
    lse = jnp.log(jnp.sum(jnp.exp(z), axis=1, keepdims=True))
    out_ref[...] = z - lse


def _epi2(m2, e2, h2, as2, ad2, kv2, b2, r16):
    return pl.pallas_call(
        _epi2_body,
        grid=(N // BM,),
        in_specs=[
            pl.BlockSpec((2, BM, HC), lambda i: (0, i, 0)),
            pl.BlockSpec((2, BM, 16), lambda i: (0, i, 0)),
            pl.BlockSpec((BM, HC), lambda i: (i, 0)),
            pl.BlockSpec((BM, 16), lambda i: (i, 0)),
            pl.BlockSpec((BM, 16), lambda i: (i, 0)),
            pl.BlockSpec((1, 16), lambda i: (0, 0)),
            pl.BlockSpec((1, HC), lambda i: (0, 0)),
            pl.BlockSpec((16, HC), lambda i: (0, 0)),
        ],
        out_specs=pl.BlockSpec((BM, HC), lambda i: (i, 0)),
        out_shape=jax.ShapeDtypeStruct((N, HC), jnp.float32),
    )(m2, e2, h2, as2, ad2, kv2, b2, r16)


def _pack_mats(att_src, att_dst, ch):
    fs = att_src.reshape(HC)
    fd = att_dst.reshape(HC)
    rows = jnp.arange(HC)
    cols = jnp.arange(16)
    if ch == HC:
        sel = jnp.ones((HC, 16), jnp.float32)
    else:
        sel = (cols[None, :] == (rows[:, None] // ch)).astype(jnp.float32)
    return fs[:, None] * sel, fd[:, None] * sel


def _bcast_mat(ch):
    return ((jnp.arange(HC)[None, :] // ch) == jnp.arange(16)[:, None]).astype(
        jnp.float32)


def kernel(x, edge_index, W1, att_src1, att_dst1, b1, W2, att_src2, att_dst2,
           b2):
    asm1, adm1 = _pack_mats(att_src1, att_dst1, 8)
    asm2, adm2 = _pack_mats(att_src2, att_dst2, HC)
    r16_1 = _bcast_mat(8)
    r16_2 = _bcast_mat(HC)
    npad_e = EPAD - E
    srcp = jnp.concatenate([edge_index[0], jnp.zeros((npad_e,), jnp.int32)])
    dstp = jnp.concatenate([edge_index[1], jnp.full((npad_e,), N, jnp.int32)])
    z64 = jnp.zeros((NPAD, HC), jnp.float32)
    z16 = jnp.zeros((NPAD, 16), jnp.float32)
    b1r = b1.reshape(1, HC)
    b2r = b2.reshape(1, HC)

    h1, as1, ad1, mx1 = _dense1(x, W1, asm1, adm1)
    kv1 = _leaky(mx1[0] + mx1[1]).reshape(1, 16)
    m1, e1 = _get_edge_kernel(8)(h1, as1, ad1, srcp, dstp, kv1.reshape(16),
                                 z64, z16)

    h2, as2, ad2, mx2 = _epi1(m1, e1, h1, as1, ad1, kv1, b1r, r16_1, W2, asm2,
                              adm2)
    kv2 = _leaky(mx2[0] + mx2[1]).reshape(1, 16)
    m2, e2 = _get_edge_kernel(1)(h2, as2, ad2, srcp, dstp, kv2.reshape(16),
                                 z64, z16)

    return _epi2(m2, e2, h2, as2, ad2, kv2, b2r, r16_2)
